# Initial kernel scaffold; baseline (speedup 1.0000x reference)
import jax
import jax.numpy as jnp
from jax.experimental import pallas as pl


def kernel(x, emb_table, pos_table):
    raise NotImplementedError("write your pallas kernel here")


# speedup vs baseline: 2.2848x; 2.2848x over previous
import functools

import jax
import jax.numpy as jnp
from jax import lax
from jax.experimental import pallas as pl
from jax.experimental.pallas import tpu as pltpu
from jax.experimental.pallas import tpu_sc as plsc

NVOCAB = 100000
NHID = 64
MAXLEN = 200
BATCH = 4096
SEQ = 200

NUM_CORES = 2
NUM_SUBCORES = 16
NW = NUM_CORES * NUM_SUBCORES
SEQ_PER_W = BATCH // NW
LANES = 16

_mesh = plsc.VectorSubcoreMesh(core_axis_name="c", subcore_axis_name="s")


@functools.partial(
    pl.kernel,
    out_type=jax.ShapeDtypeStruct((BATCH * SEQ, NHID), jnp.float32),
    mesh=_mesh,
    scratch_types=[
        pltpu.VMEM((SEQ,), jnp.int32),
        pltpu.VMEM((SEQ, NHID), jnp.float32),
        pltpu.VMEM((SEQ, NHID), jnp.float32),
        pltpu.SemaphoreType.DMA,
    ],
    compiler_params=pltpu.CompilerParams(use_tc_tiling_on_sc=False),
)
def _embed(x_hbm, emb_hbm, pos_hbm, out_hbm, idx_v, rows_v, pos_v, sem):
    wid = lax.axis_index("s") * NUM_CORES + lax.axis_index("c")
    base = wid * (SEQ_PER_W * SEQ)

    pltpu.sync_copy(pos_hbm, pos_v)

    def per_seq(s, carry):
        row0 = base + s * SEQ
        pltpu.sync_copy(x_hbm.at[pl.ds(row0, SEQ)], idx_v)
        pltpu.async_copy(emb_hbm.at[idx_v], rows_v, sem).wait()

        def add_rows(r, c2):
            for c in range(NHID // LANES):
                rows_v[r, pl.ds(LANES * c, LANES)] += pos_v[r, pl.ds(LANES * c, LANES)]
            return c2

        lax.fori_loop(0, SEQ, add_rows, 0, unroll=4)
        pltpu.sync_copy(rows_v, out_hbm.at[pl.ds(row0, SEQ)])
        return carry

    lax.fori_loop(0, SEQ_PER_W, per_seq, 0)


def kernel(x, emb_table, pos_table):
    xf = x.reshape(-1).astype(jnp.int32)
    out = _embed(xf, emb_table, pos_table)
    return out.reshape(BATCH, SEQ, NHID)


# speedup vs baseline: 2.8966x; 1.2678x over previous
import functools

import jax
import jax.numpy as jnp
from jax import lax
from jax.experimental import pallas as pl
from jax.experimental.pallas import tpu as pltpu
from jax.experimental.pallas import tpu_sc as plsc

NVOCAB = 100000
NHID = 64
MAXLEN = 200
BATCH = 4096
SEQ = 200

NUM_CORES = 2
NUM_SUBCORES = 16
NW = NUM_CORES * NUM_SUBCORES
SEQ_PER_W = BATCH // NW
LANES = 16
NBUF = 4
LOOKAHEAD = 2

_mesh = plsc.VectorSubcoreMesh(core_axis_name="c", subcore_axis_name="s")


@functools.partial(
    pl.kernel,
    out_type=jax.ShapeDtypeStruct((BATCH * SEQ, NHID), jnp.float32),
    mesh=_mesh,
    scratch_types=[
        pltpu.VMEM((SEQ_PER_W * SEQ,), jnp.int32),
        pltpu.VMEM((SEQ, NHID), jnp.float32),
        [pltpu.VMEM((SEQ, NHID), jnp.float32) for _ in range(NBUF)],
        [pltpu.SemaphoreType.DMA for _ in range(NBUF)],
        [pltpu.SemaphoreType.DMA for _ in range(NBUF)],
    ],
    compiler_params=pltpu.CompilerParams(use_tc_tiling_on_sc=False),
)
def _embed(x_hbm, emb_hbm, pos_hbm, out_hbm, idx_all, pos_v, rows, gsem, ssem):
    wid = lax.axis_index("s") * NUM_CORES + lax.axis_index("c")
    base = wid * (SEQ_PER_W * SEQ)

    pltpu.sync_copy(x_hbm.at[pl.ds(base, SEQ_PER_W * SEQ)], idx_all)
    pltpu.sync_copy(pos_hbm, pos_v)

    def gather_desc(g, b):
        src = emb_hbm.at[idx_all.at[pl.ds(g * SEQ, SEQ)]]
        return pltpu.make_async_copy(src, rows[b], gsem[b])

    def store_desc(g, b):
        return pltpu.make_async_copy(rows[b], out_hbm.at[pl.ds(base + g * SEQ, SEQ)], ssem[b])

    for b in range(LOOKAHEAD):
        gather_desc(b, b).start()

    def step(t, carry):
        for j in range(NBUF):
            g = t * NBUF + j
            nb = (j + LOOKAHEAD) % NBUF
            ng = g + LOOKAHEAD

            @pl.when(ng < SEQ_PER_W)
            def _():
                @pl.when(ng >= NBUF)
                def _():
                    store_desc(ng - NBUF, nb).wait()
                gather_desc(ng, nb).start()

            gather_desc(g, j).wait()

            def add_rows(r, c2, _rows=rows[j]):
                for c in range(NHID // LANES):
                    sl = pl.ds(LANES * c, LANES)
                    _rows[r, sl] += pos_v[r, sl]
                return c2

            lax.fori_loop(0, SEQ, add_rows, 0, unroll=4)
            store_desc(g, j).start()
        return carry

    lax.fori_loop(0, SEQ_PER_W // NBUF, step, 0)

    for k in range(LOOKAHEAD):
        g = SEQ_PER_W - LOOKAHEAD + k
        store_desc(g, g % NBUF).wait()


def kernel(x, emb_table, pos_table):
    xf = x.reshape(-1).astype(jnp.int32)
    out = _embed(xf, emb_table, pos_table)
    return out.reshape(BATCH, SEQ, NHID)
